# baseline (speedup 1.0000x reference)
import numpy as np
import jax
import jax.numpy as jnp
from jax import lax
from jax.experimental import pallas as pl
from jax.experimental.pallas import tpu as pltpu
from jax.experimental.pallas import tpu_sc as plsc

_DIM = 12
_NCLASS = 2 ** _DIM
_SPAN = _NCLASS - _DIM - 1
_MULT = (2 ** 16 % _SPAN) ** 2 % _SPAN

_NC, _NS, _L = 2, 16, 16

_ROTS = ((13, 15, 26, 6), (17, 29, 16, 24))


def _np_threefry2x32(k0, k1, x0, x1):
    ks = [np.uint32(k0), np.uint32(k1),
          np.uint32(k0) ^ np.uint32(k1) ^ np.uint32(0x1BD11BDA)]
    x0 = (x0 + ks[0]).astype(np.uint32)
    x1 = (x1 + ks[1]).astype(np.uint32)
    for i in range(5):
        for r in _ROTS[i % 2]:
            x0 = (x0 + x1).astype(np.uint32)
            x1 = ((x1 << np.uint32(r)) | (x1 >> np.uint32(32 - r))).astype(np.uint32)
            x1 = (x1 ^ x0).astype(np.uint32)
        x0 = (x0 + ks[(i + 1) % 3]).astype(np.uint32)
        x1 = (x1 + ks[(i + 2) % 3] + np.uint32(i + 1)).astype(np.uint32)
    return x0, x1


_SA, _SB = _np_threefry2x32(np.uint32(0), np.uint32(42),
                            np.zeros(2, dtype=np.uint32),
                            np.arange(2, dtype=np.uint32))
_KEY_U = (int(_SA[0]), int(_SB[0]))
_KEY_V = (int(_SA[1]), int(_SB[1]))


def _tf_bits(key, x0, x1):
    k0, k1 = np.uint32(key[0]), np.uint32(key[1])
    k2 = np.uint32(k0 ^ k1 ^ np.uint32(0x1BD11BDA))
    x0 = x0 + jnp.uint32(k0)
    x1 = x1 + jnp.uint32(k1)

    def group(g, carry):
        x0, x1, p, q, s = carry
        even = (g % 2) == 0
        for r0, r1 in zip(*_ROTS):
            r = jnp.where(even, jnp.uint32(r0), jnp.uint32(r1))
            x0 = x0 + x1
            x1 = (x1 << r) | (x1 >> (jnp.uint32(32) - r))
            x1 = x1 ^ x0
        x0 = x0 + p
        x1 = x1 + q + (g + 1).astype(jnp.uint32)
        return (x0, x1, q, s, p)

    x0, x1, _, _, _ = lax.fori_loop(
        0, 5, group,
        (x0, x1, jnp.uint32(k1), jnp.uint32(k2), jnp.uint32(k0)))
    return x0 ^ x1


def _sc_sample_gather(action_map, batch):
    n_workers = _NC * _NS
    per_w = batch // n_workers
    n_vec = per_w // _L
    mesh = plsc.VectorSubcoreMesh(core_axis_name="c", subcore_axis_name="s")

    def body(am_hbm, r_hbm, idx_v, vals_v, sem):
        wid = lax.axis_index("c") * _NS + lax.axis_index("s")
        base = wid * per_w
        lanes = lax.iota(jnp.int32, _L)
        span = jnp.uint32(_SPAN)
        mult = jnp.uint32(_MULT)

        def jbody(j, carry):
            lo = (lanes + (base + j * _L)).astype(jnp.uint32)
            hi = jnp.zeros((_L,), jnp.uint32)
            u = _tf_bits(_KEY_U, hi, lo)
            v = _tf_bits(_KEY_V, hi, lo)
            off = ((u % span) * mult + (v % span)) % span
            idx_v[pl.ds(j * _L, _L)] = off.astype(jnp.int32)
            return carry

        lax.fori_loop(0, n_vec, jbody, 0)
        pltpu.async_copy(am_hbm.at[idx_v], vals_v, sem).wait()
        pltpu.sync_copy(vals_v, r_hbm.at[pl.ds(base, per_w)])

    return pl.kernel(
        body,
        out_type=jax.ShapeDtypeStruct((batch,), jnp.float32),
        mesh=mesh,
        compiler_params=pltpu.CompilerParams(needs_layout_passes=False),
        scratch_types=[
            pltpu.VMEM((per_w,), jnp.int32),
            pltpu.VMEM((per_w,), jnp.float32),
            pltpu.SemaphoreType.DMA,
        ],
    )(action_map)


_BLK = 256


def _onehot_body(r_ref, o_ref):
    r4 = r_ref[0]
    cols = lax.broadcasted_iota(jnp.int32, (128, _NCLASS), 1)
    for a in range(_BLK // 128):
        rt = jnp.transpose(r4[a:a + 1, :], (1, 0)).astype(jnp.int32)
        o_ref[pl.ds(a * 128, 128), :] = (cols == rt).astype(jnp.float32)


def kernel(x, action_map):
    batch = x.shape[0]
    r = _sc_sample_gather(action_map, batch)
    out = pl.pallas_call(
        _onehot_body,
        grid=(batch // _BLK,),
        in_specs=[pl.BlockSpec((1, _BLK // 128, 128), lambda i: (i, 0, 0))],
        out_specs=pl.BlockSpec((_BLK, _NCLASS), lambda i: (i, 0)),
        out_shape=jax.ShapeDtypeStruct((batch, _NCLASS), jnp.float32),
    )(r.reshape(batch // _BLK, _BLK // 128, 128))
    return out


# speedup vs baseline: 1.0114x; 1.0114x over previous
import numpy as np
import jax
import jax.numpy as jnp
from jax import lax
from jax.experimental import pallas as pl
from jax.experimental.pallas import tpu as pltpu
from jax.experimental.pallas import tpu_sc as plsc

_DIM = 12
_NCLASS = 2 ** _DIM
_SPAN = _NCLASS - _DIM - 1
_MULT = (2 ** 16 % _SPAN) ** 2 % _SPAN

_NC, _NS, _L = 2, 16, 16

_ROTS = ((13, 15, 26, 6), (17, 29, 16, 24))


def _np_threefry2x32(k0, k1, x0, x1):
    ks = [np.uint32(k0), np.uint32(k1),
          np.uint32(k0) ^ np.uint32(k1) ^ np.uint32(0x1BD11BDA)]
    x0 = (x0 + ks[0]).astype(np.uint32)
    x1 = (x1 + ks[1]).astype(np.uint32)
    for i in range(5):
        for r in _ROTS[i % 2]:
            x0 = (x0 + x1).astype(np.uint32)
            x1 = ((x1 << np.uint32(r)) | (x1 >> np.uint32(32 - r))).astype(np.uint32)
            x1 = (x1 ^ x0).astype(np.uint32)
        x0 = (x0 + ks[(i + 1) % 3]).astype(np.uint32)
        x1 = (x1 + ks[(i + 2) % 3] + np.uint32(i + 1)).astype(np.uint32)
    return x0, x1


_SA, _SB = _np_threefry2x32(np.uint32(0), np.uint32(42),
                            np.zeros(2, dtype=np.uint32),
                            np.arange(2, dtype=np.uint32))
_KEY_U = (int(_SA[0]), int(_SB[0]))
_KEY_V = (int(_SA[1]), int(_SB[1]))


def _tf_bits(key, x0, x1):
    k0, k1 = np.uint32(key[0]), np.uint32(key[1])
    k2 = np.uint32(k0 ^ k1 ^ np.uint32(0x1BD11BDA))
    x0 = x0 + jnp.uint32(k0)
    x1 = x1 + jnp.uint32(k1)

    def group(g, carry):
        x0, x1, p, q, s = carry
        even = (g % 2) == 0
        for r0, r1 in zip(*_ROTS):
            r = jnp.where(even, jnp.uint32(r0), jnp.uint32(r1))
            x0 = x0 + x1
            x1 = (x1 << r) | (x1 >> (jnp.uint32(32) - r))
            x1 = x1 ^ x0
        x0 = x0 + p
        x1 = x1 + q + (g + 1).astype(jnp.uint32)
        return (x0, x1, q, s, p)

    x0, x1, _, _, _ = lax.fori_loop(
        0, 5, group,
        (x0, x1, jnp.uint32(k1), jnp.uint32(k2), jnp.uint32(k0)))
    return x0 ^ x1


def _sc_sample_gather(action_map, batch):
    n_workers = _NC * _NS
    per_w = batch // n_workers
    n_vec = per_w // _L
    mesh = plsc.VectorSubcoreMesh(core_axis_name="c", subcore_axis_name="s")

    def body(am_hbm, r_hbm, vals_v, sem):
        wid = lax.axis_index("c") * _NS + lax.axis_index("s")
        base = wid * per_w
        lanes = lax.iota(jnp.int32, _L)
        span = jnp.uint32(_SPAN)
        mult = jnp.uint32(_MULT)

        def jbody(j, carry):
            lo = (lanes + (base + j * _L)).astype(jnp.uint32)
            hi = jnp.zeros((_L,), jnp.uint32)
            u = _tf_bits(_KEY_U, hi, lo)
            v = _tf_bits(_KEY_V, hi, lo)
            off = ((u % span) * mult + (v % span)) % span
            pltpu.async_copy(am_hbm.at[off.astype(jnp.int32)],
                             vals_v.at[pl.ds(j * _L, _L)], sem)
            return carry

        lax.fori_loop(0, n_vec, jbody, 0)
        for _ in range(n_vec):
            pltpu.make_async_copy(am_hbm.at[pl.ds(0, _L)],
                                  vals_v.at[pl.ds(0, _L)], sem).wait()
        pltpu.sync_copy(vals_v, r_hbm.at[pl.ds(base, per_w)])

    return pl.kernel(
        body,
        out_type=jax.ShapeDtypeStruct((batch,), jnp.float32),
        mesh=mesh,
        compiler_params=pltpu.CompilerParams(needs_layout_passes=False),
        scratch_types=[
            pltpu.VMEM((per_w,), jnp.float32),
            pltpu.SemaphoreType.DMA,
        ],
    )(action_map)


_BLK = 256


def _onehot_body(r_ref, o_ref):
    r4 = r_ref[0]
    cols = lax.broadcasted_iota(jnp.int32, (128, _NCLASS), 1)
    for a in range(_BLK // 128):
        rt = jnp.transpose(r4[a:a + 1, :], (1, 0)).astype(jnp.int32)
        o_ref[pl.ds(a * 128, 128), :] = (cols == rt).astype(jnp.float32)


def kernel(x, action_map):
    batch = x.shape[0]
    r = _sc_sample_gather(action_map, batch)
    out = pl.pallas_call(
        _onehot_body,
        grid=(batch // _BLK,),
        in_specs=[pl.BlockSpec((1, _BLK // 128, 128), lambda i: (i, 0, 0))],
        out_specs=pl.BlockSpec((_BLK, _NCLASS), lambda i: (i, 0)),
        out_shape=jax.ShapeDtypeStruct((batch, _NCLASS), jnp.float32),
    )(r.reshape(batch // _BLK, _BLK // 128, 128))
    return out
